# baseline (speedup 1.0000x reference)
import functools

import jax
import jax.numpy as jnp
from jax import lax
from jax.experimental import pallas as pl
from jax.experimental.pallas import tpu as pltpu
from jax.experimental.pallas import tpu_sc as plsc

N = 10000
NP = 10240
E = 320000
IN_FEATS = 128
HF = IN_FEATS // 2
NUM_CLASSES = 16
NC = 2
NS = 16
NW = NC * NS
EPW = E // NW
CHUNK = 128
NCH = 80
EPWP = NCH * CHUNK
NBUF = 2
RPT = NP // NS
BR = 320
GRID = NP // BR

_mesh = plsc.VectorSubcoreMesh(core_axis_name="c", subcore_axis_name="s")
_params = pltpu.CompilerParams(use_tc_tiling_on_sc=False)


def _fill(ref, rows, cols, val):
    v = jnp.full((16,), val, jnp.float32)

    def outer(i, c):
        def inner(k, c2):
            ref[i, pl.ds(k * 16, 16)] = v
            return c2

        return lax.fori_loop(0, cols // 16, inner, c)

    lax.fori_loop(0, rows, outer, 0)


@functools.partial(
    pl.kernel,
    out_type=[
        jax.ShapeDtypeStruct((NC, NP, 16), jnp.float32),
        jax.ShapeDtypeStruct((NC, NP, 16), jnp.float32),
    ],
    mesh=_mesh,
    compiler_params=_params,
    scratch_types=[
        pltpu.VMEM((NCH, CHUNK), jnp.int32),
        pltpu.VMEM((CHUNK, 16), jnp.float32),
        pltpu.VMEM((CHUNK, 16), jnp.float32),
        pltpu.VMEM_SHARED((NP, 16), jnp.float32),
        pltpu.VMEM_SHARED((NP, 16), jnp.float32),
    ],
)
def _degrees(src_hbm, dst_hbm, dout_hbm, din_hbm, idx_v, ones_v, zeros_v, acc_o, acc_i):
    c = lax.axis_index("c")
    s = lax.axis_index("s")
    w = c * NS + s
    _fill(ones_v, CHUNK, 16, 1.0)
    _fill(zeros_v, CHUNK, 16, 0.0)
    base = s * RPT

    def zacc(k, carry):
        pltpu.sync_copy(zeros_v, acc_o.at[pl.ds(base + k * CHUNK, CHUNK)])
        pltpu.sync_copy(zeros_v, acc_i.at[pl.ds(base + k * CHUNK, CHUNK)])
        return carry

    lax.fori_loop(0, RPT // CHUNK, zacc, 0)
    plsc.subcore_barrier()

    pltpu.sync_copy(src_hbm.at[w], idx_v)

    def scat_o(j, carry):
        pltpu.sync_copy(ones_v, acc_o.at[idx_v.at[j]], add=True)
        return carry

    lax.fori_loop(0, NCH, scat_o, 0)
    pltpu.sync_copy(dst_hbm.at[w], idx_v)

    def scat_i(j, carry):
        pltpu.sync_copy(ones_v, acc_i.at[idx_v.at[j]], add=True)
        return carry

    lax.fori_loop(0, NCH, scat_i, 0)
    plsc.subcore_barrier()

    pltpu.sync_copy(acc_o.at[pl.ds(base, RPT)], dout_hbm.at[c, pl.ds(base, RPT)])
    pltpu.sync_copy(acc_i.at[pl.ds(base, RPT)], din_hbm.at[c, pl.ds(base, RPT)])


def _make_propagate(D, CHUNK, NCH):
    @functools.partial(
        pl.kernel,
        out_type=jax.ShapeDtypeStruct((NC, NP, D), jnp.float32),
        mesh=_mesh,
        compiler_params=_params,
        scratch_types=[
            pltpu.VMEM((NCH, CHUNK), jnp.int32),
            pltpu.VMEM((NCH, CHUNK), jnp.int32),
            [pltpu.VMEM((CHUNK, D), jnp.float32) for _ in range(2 * NBUF)],
            pltpu.VMEM((CHUNK, D), jnp.float32),
            pltpu.VMEM_SHARED((NP, D), jnp.float32),
            [pltpu.SemaphoreType.DMA for _ in range(2 * NBUF)],
            [pltpu.SemaphoreType.DMA for _ in range(2 * NBUF)],
        ],
    )
    def prop(table_hbm, src_hbm, dst_hbm, out_hbm, idx_s, idx_d, bufs, zeros_v, acc,
             gsems, ssems):
        c = lax.axis_index("c")
        s = lax.axis_index("s")
        w = c * NS + s
        nb2 = 2 * NBUF
        _fill(zeros_v, CHUNK, D, 0.0)
        base = s * RPT

        def zacc(k, carry):
            pltpu.sync_copy(zeros_v, acc.at[pl.ds(base + k * CHUNK, CHUNK)])
            return carry

        lax.fori_loop(0, RPT // CHUNK, zacc, 0)
        plsc.subcore_barrier()

        pltpu.sync_copy(src_hbm.at[w], idx_s)
        pltpu.sync_copy(dst_hbm.at[w], idx_d)

        for b in range(NBUF):
            pltpu.async_copy(table_hbm.at[idx_s.at[b]], bufs[b], gsems[b])

        def ring(q, carry):
            j0 = q * nb2
            for i in range(nb2):
                j = j0 + i
                b = i
                b4 = (i + NBUF) % nb2
                pltpu.make_async_copy(table_hbm.at[idx_s.at[j]], bufs[b], gsems[b]).wait()
                pltpu.async_copy(bufs[b], acc.at[idx_d.at[j]], ssems[b], add=True)

                @pl.when(j >= NBUF)
                def _drain():
                    pltpu.make_async_copy(
                        bufs[b4], acc.at[idx_d.at[j - NBUF]], ssems[b4]
                    ).wait()

                @pl.when(j + NBUF < NCH)
                def _issue():
                    pltpu.async_copy(table_hbm.at[idx_s.at[j + NBUF]], bufs[b4], gsems[b4])

            return carry

        lax.fori_loop(0, NCH // nb2, ring, 0)
        for i in range(NBUF):
            jj = NCH - NBUF + i
            b = jj % nb2
            pltpu.make_async_copy(bufs[b], acc.at[idx_d.at[jj]], ssems[b]).wait()
        plsc.subcore_barrier()

        pltpu.sync_copy(acc.at[pl.ds(base, RPT)], out_hbm.at[c, pl.ds(base, RPT)])

    return prop


CH64 = 128
_prop64 = _make_propagate(HF, CH64, EPWP // CH64)
_prop16 = _make_propagate(NUM_CLASSES, CHUNK, NCH)


def _norm(dp_block):
    deg = dp_block[0, :, 0] + dp_block[1, :, 0]
    return lax.rsqrt(jnp.where(deg > 0, deg, 1.0))


def _tc_prep(xp, doutp, W1):
    def body(x_ref, d_ref, w_ref, ya_ref, yb_ref):
        nsrc = _norm(d_ref[...])
        y = jnp.dot(
            x_ref[...] * nsrc[:, None], w_ref[...], preferred_element_type=jnp.float32
        )
        ya_ref[...] = y[:, :HF]
        yb_ref[...] = y[:, HF:]

    return pl.pallas_call(
        body,
        grid=(GRID,),
        in_specs=[
            pl.BlockSpec((BR, IN_FEATS), lambda i: (i, 0)),
            pl.BlockSpec((NC, BR, 16), lambda i: (0, i, 0)),
            pl.BlockSpec((IN_FEATS, IN_FEATS), lambda i: (0, 0)),
        ],
        out_specs=[
            pl.BlockSpec((BR, HF), lambda i: (i, 0)),
            pl.BlockSpec((BR, HF), lambda i: (i, 0)),
        ],
        out_shape=[
            jax.ShapeDtypeStruct((NP, HF), jnp.float32),
            jax.ShapeDtypeStruct((NP, HF), jnp.float32),
        ],
    )(xp, doutp, W1)


def _tc_mid(Pa, Pb, doutp, dinp, b1, W2):
    def body(pa_ref, pb_ref, do_ref, di_ref, b_ref, w_ref, y_ref):
        ndst = _norm(di_ref[...])
        nsrc = _norm(do_ref[...])
        b = b_ref[...]
        w = w_ref[...]
        ha = jnp.maximum((pa_ref[0] + pa_ref[1]) * ndst[:, None] + b[:, :HF], 0.0)
        hb = jnp.maximum((pb_ref[0] + pb_ref[1]) * ndst[:, None] + b[:, HF:], 0.0)
        y_ref[...] = jnp.dot(
            ha * nsrc[:, None], w[:HF], preferred_element_type=jnp.float32
        ) + jnp.dot(hb * nsrc[:, None], w[HF:], preferred_element_type=jnp.float32)

    return pl.pallas_call(
        body,
        grid=(GRID,),
        in_specs=[
            pl.BlockSpec((NC, BR, HF), lambda i: (0, i, 0)),
            pl.BlockSpec((NC, BR, HF), lambda i: (0, i, 0)),
            pl.BlockSpec((NC, BR, 16), lambda i: (0, i, 0)),
            pl.BlockSpec((NC, BR, 16), lambda i: (0, i, 0)),
            pl.BlockSpec((1, IN_FEATS), lambda i: (0, 0)),
            pl.BlockSpec((IN_FEATS, NUM_CLASSES), lambda i: (0, 0)),
        ],
        out_specs=pl.BlockSpec((BR, NUM_CLASSES), lambda i: (i, 0)),
        out_shape=jax.ShapeDtypeStruct((NP, NUM_CLASSES), jnp.float32),
    )(Pa, Pb, doutp, dinp, b1, W2)


def _tc_fin(Q, dinp, b2):
    def body(q_ref, di_ref, b_ref, o_ref):
        ndst = _norm(di_ref[...])
        o_ref[...] = (q_ref[0] + q_ref[1]) * ndst[:, None] + b_ref[...]

    return pl.pallas_call(
        body,
        grid=(GRID,),
        in_specs=[
            pl.BlockSpec((NC, BR, NUM_CLASSES), lambda i: (0, i, 0)),
            pl.BlockSpec((NC, BR, 16), lambda i: (0, i, 0)),
            pl.BlockSpec((1, NUM_CLASSES), lambda i: (0, 0)),
        ],
        out_specs=pl.BlockSpec((BR, NUM_CLASSES), lambda i: (i, 0)),
        out_shape=jax.ShapeDtypeStruct((NP, NUM_CLASSES), jnp.float32),
    )(Q, dinp, b2)


def kernel(x, edge_index, W1, b1, W2, b2):
    src = edge_index[0].reshape(NW, EPW)
    dst = edge_index[1].reshape(NW, EPW)
    srcp = jnp.pad(src, ((0, 0), (0, EPWP - EPW)), constant_values=N)
    srcp = srcp.reshape(NW, NCH, CHUNK)
    dstp = jnp.pad(dst, ((0, 0), (0, EPWP - EPW)), constant_values=NP - 1)
    dstp = dstp.reshape(NW, NCH, CHUNK)
    xp = jnp.pad(x, ((0, NP - N), (0, 0)))

    srcp64 = srcp.reshape(NW, EPWP // CH64, CH64)
    dstp64 = dstp.reshape(NW, EPWP // CH64, CH64)
    doutp, dinp = _degrees(srcp, dstp)
    y1a, y1b = _tc_prep(xp, doutp, W1)
    Pa = _prop64(y1a, srcp64, dstp64)
    Pb = _prop64(y1b, srcp64, dstp64)
    y2 = _tc_mid(Pa, Pb, doutp, dinp, b1.reshape(1, IN_FEATS), W2)
    Q = _prop16(y2, srcp, dstp)
    out = _tc_fin(Q, dinp, b2.reshape(1, NUM_CLASSES))
    return out[:N]


# speedup vs baseline: 1.0528x; 1.0528x over previous
import functools

import jax
import jax.numpy as jnp
from jax import lax
from jax.experimental import pallas as pl
from jax.experimental.pallas import tpu as pltpu
from jax.experimental.pallas import tpu_sc as plsc

N = 10000
NP = 10240
E = 320000
IN_FEATS = 128
HF = IN_FEATS // 2
NUM_CLASSES = 16
NC = 2
NS = 16
NW = NC * NS
EPW = E // NW
CHUNK = 128
NCH = 80
EPWP = NCH * CHUNK
NBUF = 4
RPT = NP // NS
BR = 320
GRID = NP // BR

_mesh = plsc.VectorSubcoreMesh(core_axis_name="c", subcore_axis_name="s")
_params = pltpu.CompilerParams(use_tc_tiling_on_sc=False)


def _fill(ref, rows, cols, val):
    v = jnp.full((16,), val, jnp.float32)

    def outer(i, c):
        def inner(k, c2):
            ref[i, pl.ds(k * 16, 16)] = v
            return c2

        return lax.fori_loop(0, cols // 16, inner, c)

    lax.fori_loop(0, rows, outer, 0)


@functools.partial(
    pl.kernel,
    out_type=[
        jax.ShapeDtypeStruct((NC, NP, 16), jnp.float32),
        jax.ShapeDtypeStruct((NC, NP, 16), jnp.float32),
    ],
    mesh=_mesh,
    compiler_params=_params,
    scratch_types=[
        pltpu.VMEM((NCH, CHUNK), jnp.int32),
        pltpu.VMEM((CHUNK, 16), jnp.float32),
        pltpu.VMEM((CHUNK, 16), jnp.float32),
        pltpu.VMEM_SHARED((NP, 16), jnp.float32),
        pltpu.VMEM_SHARED((NP, 16), jnp.float32),
    ],
)
def _degrees(src_hbm, dst_hbm, dout_hbm, din_hbm, idx_v, ones_v, zeros_v, acc_o, acc_i):
    c = lax.axis_index("c")
    s = lax.axis_index("s")
    w = c * NS + s
    _fill(ones_v, CHUNK, 16, 1.0)
    _fill(zeros_v, CHUNK, 16, 0.0)
    base = s * RPT

    def zacc(k, carry):
        pltpu.sync_copy(zeros_v, acc_o.at[pl.ds(base + k * CHUNK, CHUNK)])
        pltpu.sync_copy(zeros_v, acc_i.at[pl.ds(base + k * CHUNK, CHUNK)])
        return carry

    lax.fori_loop(0, RPT // CHUNK, zacc, 0)
    plsc.subcore_barrier()

    pltpu.sync_copy(src_hbm.at[w], idx_v)

    def scat_o(j, carry):
        pltpu.sync_copy(ones_v, acc_o.at[idx_v.at[j]], add=True)
        return carry

    lax.fori_loop(0, NCH, scat_o, 0)
    pltpu.sync_copy(dst_hbm.at[w], idx_v)

    def scat_i(j, carry):
        pltpu.sync_copy(ones_v, acc_i.at[idx_v.at[j]], add=True)
        return carry

    lax.fori_loop(0, NCH, scat_i, 0)
    plsc.subcore_barrier()

    pltpu.sync_copy(acc_o.at[pl.ds(base, RPT)], dout_hbm.at[c, pl.ds(base, RPT)])
    pltpu.sync_copy(acc_i.at[pl.ds(base, RPT)], din_hbm.at[c, pl.ds(base, RPT)])


def _make_propagate(D, CHUNK, NCH):
    @functools.partial(
        pl.kernel,
        out_type=jax.ShapeDtypeStruct((NC, NP, D), jnp.float32),
        mesh=_mesh,
        compiler_params=_params,
        scratch_types=[
            pltpu.VMEM((NCH, CHUNK), jnp.int32),
            pltpu.VMEM((NCH, CHUNK), jnp.int32),
            [pltpu.VMEM((CHUNK, D), jnp.float32) for _ in range(NBUF)],
            pltpu.VMEM((CHUNK, D), jnp.float32),
            pltpu.VMEM_SHARED((NP, D), jnp.float32),
            [pltpu.SemaphoreType.DMA for _ in range(NBUF)],
        ],
    )
    def prop(table_hbm, src_hbm, dst_hbm, out_hbm, idx_s, idx_d, bufs, zeros_v, acc, sems):
        c = lax.axis_index("c")
        s = lax.axis_index("s")
        w = c * NS + s
        _fill(zeros_v, CHUNK, D, 0.0)
        base = s * RPT

        def zacc(k, carry):
            pltpu.sync_copy(zeros_v, acc.at[pl.ds(base + k * CHUNK, CHUNK)])
            return carry

        lax.fori_loop(0, RPT // CHUNK, zacc, 0)
        plsc.subcore_barrier()

        pltpu.sync_copy(src_hbm.at[w], idx_s)
        pltpu.sync_copy(dst_hbm.at[w], idx_d)

        for b in range(NBUF):
            pltpu.async_copy(table_hbm.at[idx_s.at[b]], bufs[b], sems[b])

        def ring(q, carry):
            j0 = q * NBUF
            for b in range(NBUF):
                j = j0 + b
                pltpu.make_async_copy(table_hbm.at[idx_s.at[j]], bufs[b], sems[b]).wait()
                pltpu.sync_copy(bufs[b], acc.at[idx_d.at[j]], add=True)
                nxt = j + NBUF

                @pl.when(nxt < NCH)
                def _issue():
                    pltpu.async_copy(table_hbm.at[idx_s.at[nxt]], bufs[b], sems[b])

            return carry

        lax.fori_loop(0, NCH // NBUF, ring, 0)
        plsc.subcore_barrier()

        pltpu.sync_copy(acc.at[pl.ds(base, RPT)], out_hbm.at[c, pl.ds(base, RPT)])

    return prop


CH64 = 128
_prop64 = _make_propagate(HF, CH64, EPWP // CH64)
_prop16 = _make_propagate(NUM_CLASSES, CHUNK, NCH)


def _norm(dp_block):
    deg = dp_block[0, :, 0] + dp_block[1, :, 0]
    return lax.rsqrt(jnp.where(deg > 0, deg, 1.0))


def _tc_prep(xp, doutp, W1):
    def body(x_ref, d_ref, w_ref, ya_ref, yb_ref):
        nsrc = _norm(d_ref[...])
        y = jnp.dot(
            x_ref[...] * nsrc[:, None], w_ref[...], preferred_element_type=jnp.float32
        )
        ya_ref[...] = y[:, :HF]
        yb_ref[...] = y[:, HF:]

    return pl.pallas_call(
        body,
        grid=(GRID,),
        in_specs=[
            pl.BlockSpec((BR, IN_FEATS), lambda i: (i, 0)),
            pl.BlockSpec((NC, BR, 16), lambda i: (0, i, 0)),
            pl.BlockSpec((IN_FEATS, IN_FEATS), lambda i: (0, 0)),
        ],
        out_specs=[
            pl.BlockSpec((BR, HF), lambda i: (i, 0)),
            pl.BlockSpec((BR, HF), lambda i: (i, 0)),
        ],
        out_shape=[
            jax.ShapeDtypeStruct((NP, HF), jnp.float32),
            jax.ShapeDtypeStruct((NP, HF), jnp.float32),
        ],
    )(xp, doutp, W1)


def _tc_mid(Pa, Pb, doutp, dinp, b1, W2):
    def body(pa_ref, pb_ref, do_ref, di_ref, b_ref, w_ref, y_ref):
        ndst = _norm(di_ref[...])
        nsrc = _norm(do_ref[...])
        b = b_ref[...]
        w = w_ref[...]
        ha = jnp.maximum((pa_ref[0] + pa_ref[1]) * ndst[:, None] + b[:, :HF], 0.0)
        hb = jnp.maximum((pb_ref[0] + pb_ref[1]) * ndst[:, None] + b[:, HF:], 0.0)
        y_ref[...] = jnp.dot(
            ha * nsrc[:, None], w[:HF], preferred_element_type=jnp.float32
        ) + jnp.dot(hb * nsrc[:, None], w[HF:], preferred_element_type=jnp.float32)

    return pl.pallas_call(
        body,
        grid=(GRID,),
        in_specs=[
            pl.BlockSpec((NC, BR, HF), lambda i: (0, i, 0)),
            pl.BlockSpec((NC, BR, HF), lambda i: (0, i, 0)),
            pl.BlockSpec((NC, BR, 16), lambda i: (0, i, 0)),
            pl.BlockSpec((NC, BR, 16), lambda i: (0, i, 0)),
            pl.BlockSpec((1, IN_FEATS), lambda i: (0, 0)),
            pl.BlockSpec((IN_FEATS, NUM_CLASSES), lambda i: (0, 0)),
        ],
        out_specs=pl.BlockSpec((BR, NUM_CLASSES), lambda i: (i, 0)),
        out_shape=jax.ShapeDtypeStruct((NP, NUM_CLASSES), jnp.float32),
    )(Pa, Pb, doutp, dinp, b1, W2)


def _tc_fin(Q, dinp, b2):
    def body(q_ref, di_ref, b_ref, o_ref):
        ndst = _norm(di_ref[...])
        o_ref[...] = (q_ref[0] + q_ref[1]) * ndst[:, None] + b_ref[...]

    return pl.pallas_call(
        body,
        grid=(GRID,),
        in_specs=[
            pl.BlockSpec((NC, BR, NUM_CLASSES), lambda i: (0, i, 0)),
            pl.BlockSpec((NC, BR, 16), lambda i: (0, i, 0)),
            pl.BlockSpec((1, NUM_CLASSES), lambda i: (0, 0)),
        ],
        out_specs=pl.BlockSpec((BR, NUM_CLASSES), lambda i: (i, 0)),
        out_shape=jax.ShapeDtypeStruct((NP, NUM_CLASSES), jnp.float32),
    )(Q, dinp, b2)


def kernel(x, edge_index, W1, b1, W2, b2):
    src = edge_index[0].reshape(NW, EPW)
    dst = edge_index[1].reshape(NW, EPW)
    srcp = jnp.pad(src, ((0, 0), (0, EPWP - EPW)), constant_values=N)
    srcp = srcp.reshape(NW, NCH, CHUNK)
    dstp = jnp.pad(dst, ((0, 0), (0, EPWP - EPW)), constant_values=NP - 1)
    dstp = dstp.reshape(NW, NCH, CHUNK)
    xp = jnp.pad(x, ((0, NP - N), (0, 0)))

    srcp64 = srcp.reshape(NW, EPWP // CH64, CH64)
    dstp64 = dstp.reshape(NW, EPWP // CH64, CH64)
    doutp, dinp = _degrees(srcp, dstp)
    y1a, y1b = _tc_prep(xp, doutp, W1)
    Pa = _prop64(y1a, srcp64, dstp64)
    Pb = _prop64(y1b, srcp64, dstp64)
    y2 = _tc_mid(Pa, Pb, doutp, dinp, b1.reshape(1, IN_FEATS), W2)
    Q = _prop16(y2, srcp, dstp)
    out = _tc_fin(Q, dinp, b2.reshape(1, NUM_CLASSES))
    return out[:N]
